# baseline (speedup 1.0000x reference)
import functools

import jax
import jax.numpy as jnp
from jax import lax
from jax.experimental import pallas as pl
from jax.experimental.pallas import tpu as pltpu
from jax.experimental.pallas import tpu_sc as plsc

N = 10000
E = 320000
D = 128
H = 128
C = 40

NC = 2
NS = 16
NW = NC * NS

LANES = 128
N_PAD = 10112
DUMMY = N_PAD - 1
ROWS_PER_SUB = N_PAD // NS
E_ROWS = 2560
ROWS_PER_W = E_ROWS // NW

_mesh = plsc.VectorSubcoreMesh(core_axis_name="c", subcore_axis_name="s")


NBUF = 2
NPASS = 2
ROWS_PP = ROWS_PER_W // NPASS


@functools.partial(
    pl.kernel,
    out_type=jax.ShapeDtypeStruct((NC, N_PAD, H), jnp.float32),
    mesh=_mesh,
    scratch_types=[
        pltpu.VMEM((ROWS_PP, LANES), jnp.int32),
        pltpu.VMEM((ROWS_PP, LANES), jnp.int32),
        pltpu.VMEM((NBUF, LANES, H), jnp.float32),
        pltpu.VMEM_SHARED((N_PAD, H), jnp.float32),
    ]
    + [pltpu.SemaphoreType.DMA] * (2 * NBUF),
)
def _sc_scatter(g_hbm, src_hbm, dst_hbm, zeros_hbm, out_hbm,
                srcv, dstv, rowbuf, acc, *sems):
    gsem = sems[:NBUF]
    ssem = sems[NBUF:]
    c = lax.axis_index("c")
    s = lax.axis_index("s")
    wid = c * NS + s
    pltpu.sync_copy(zeros_hbm, acc.at[pl.ds(s * ROWS_PER_SUB, ROWS_PER_SUB)])
    plsc.subcore_barrier()

    for p in range(NPASS):
        base = wid * ROWS_PER_W + p * ROWS_PP
        pltpu.sync_copy(src_hbm.at[pl.ds(base, ROWS_PP)], srcv)
        pltpu.sync_copy(dst_hbm.at[pl.ds(base, ROWS_PP)], dstv)
        for b in range(NBUF):
            pltpu.async_copy(g_hbm.at[srcv.at[b]], rowbuf.at[b], gsem[b])

        @pl.loop(0, ROWS_PP, step=NBUF)
        def _(j):
            for b in range(NBUF):
                k = j + b
                pltpu.make_async_copy(
                    g_hbm.at[srcv.at[k]], rowbuf.at[b], gsem[b]).wait()
                pltpu.async_copy(
                    rowbuf.at[b], acc.at[dstv.at[k]], ssem[b], add=True)

                @pl.when(k + NBUF < ROWS_PP)
                def _():
                    pltpu.make_async_copy(
                        rowbuf.at[b], acc.at[dstv.at[k]], ssem[b]).wait()
                    pltpu.async_copy(
                        g_hbm.at[srcv.at[k + NBUF]], rowbuf.at[b], gsem[b])

        for b in range(NBUF):
            pltpu.make_async_copy(
                rowbuf.at[b], acc.at[dstv.at[0]], ssem[b]).wait()

    plsc.subcore_barrier()
    pltpu.sync_copy(
        acc.at[pl.ds(s * ROWS_PER_SUB, ROWS_PER_SUB)],
        out_hbm.at[c, pl.ds(s * ROWS_PER_SUB, ROWS_PER_SUB)],
    )


BLK = 1264
GRID = N_PAD // BLK


def _tc_mm(x, W):
    def body(x_ref, w_ref, o_ref):
        o_ref[...] = jnp.dot(x_ref[...], w_ref[...],
                             preferred_element_type=jnp.float32)
    return pl.pallas_call(
        body,
        grid=(GRID,),
        in_specs=[pl.BlockSpec((BLK, D), lambda i: (i, 0)),
                  pl.BlockSpec((D, H), lambda i: (0, 0))],
        out_specs=pl.BlockSpec((BLK, H), lambda i: (i, 0)),
        out_shape=jax.ShapeDtypeStruct((N_PAD, H), jnp.float32),
    )(x, W)


def _tc_prep(degp, m1):
    def body(deg_ref, m_ref, dinv_ref, g_ref):
        dinv = lax.rsqrt(deg_ref[0, :, 0:1] + deg_ref[1, :, 0:1] + 1.0)
        dinv_ref[...] = dinv
        g_ref[...] = m_ref[...] * dinv
    return pl.pallas_call(
        body,
        grid=(GRID,),
        in_specs=[pl.BlockSpec((NC, BLK, H), lambda i: (0, i, 0)),
                  pl.BlockSpec((BLK, H), lambda i: (i, 0))],
        out_specs=[pl.BlockSpec((BLK, 1), lambda i: (i, 0)),
                   pl.BlockSpec((BLK, H), lambda i: (i, 0))],
        out_shape=[jax.ShapeDtypeStruct((N_PAD, 1), jnp.float32),
                   jax.ShapeDtypeStruct((N_PAD, H), jnp.float32)],
    )(degp, m1)


def _tc_layer(parts, g_prev, dinv, b, W_next):
    def body(p_ref, g_ref, dinv_ref, b_ref, w_ref, o_ref):
        ssum = p_ref[0] + p_ref[1] + g_ref[...]
        h = jnp.tanh(dinv_ref[...] * ssum + b_ref[...])
        o_ref[...] = jnp.dot(h, w_ref[...],
                             preferred_element_type=jnp.float32) * dinv_ref[...]
    return pl.pallas_call(
        body,
        grid=(GRID,),
        in_specs=[pl.BlockSpec((NC, BLK, H), lambda i: (0, i, 0)),
                  pl.BlockSpec((BLK, H), lambda i: (i, 0)),
                  pl.BlockSpec((BLK, 1), lambda i: (i, 0)),
                  pl.BlockSpec((1, H), lambda i: (0, 0)),
                  pl.BlockSpec((H, H), lambda i: (0, 0))],
        out_specs=pl.BlockSpec((BLK, H), lambda i: (i, 0)),
        out_shape=jax.ShapeDtypeStruct((N_PAD, H), jnp.float32),
    )(parts, g_prev, dinv, b, W_next)


def _tc_final(parts, g_prev, dinv, b, Wc_pad, bc_pad):
    def body(p_ref, g_ref, dinv_ref, b_ref, wc_ref, bc_ref, h_ref, o_ref):
        ssum = p_ref[0] + p_ref[1] + g_ref[...]
        h = jnp.tanh(dinv_ref[...] * ssum + b_ref[...])
        h_ref[...] = h
        o_ref[...] = jnp.dot(h, wc_ref[...],
                             preferred_element_type=jnp.float32) + bc_ref[...]
    return pl.pallas_call(
        body,
        grid=(GRID,),
        in_specs=[pl.BlockSpec((NC, BLK, H), lambda i: (0, i, 0)),
                  pl.BlockSpec((BLK, H), lambda i: (i, 0)),
                  pl.BlockSpec((BLK, 1), lambda i: (i, 0)),
                  pl.BlockSpec((1, H), lambda i: (0, 0)),
                  pl.BlockSpec((H, H), lambda i: (0, 0)),
                  pl.BlockSpec((1, H), lambda i: (0, 0))],
        out_specs=[pl.BlockSpec((BLK, H), lambda i: (i, 0)),
                   pl.BlockSpec((BLK, H), lambda i: (i, 0))],
        out_shape=[jax.ShapeDtypeStruct((N_PAD, H), jnp.float32),
                   jax.ShapeDtypeStruct((N_PAD, H), jnp.float32)],
    )(parts, g_prev, dinv, b, Wc_pad, bc_pad)


@jax.jit
def kernel(x, edge_index, W1, b1, W2, b2, W3, b3, Wc, bc):
    f32 = jnp.float32
    x_pad = jnp.zeros((N_PAD, D), f32).at[:N].set(x)

    e_pad = E_ROWS * LANES - E
    src = jnp.concatenate([edge_index[0], jnp.zeros((e_pad,), jnp.int32)])
    dst = jnp.concatenate(
        [edge_index[1], jnp.full((e_pad,), DUMMY, jnp.int32)])
    src = src.reshape(E_ROWS, LANES)
    dst = dst.reshape(E_ROWS, LANES)

    ones_table = jnp.ones((N_PAD, H), f32)
    zeros_hbm = jnp.zeros((ROWS_PER_SUB, H), f32)

    Wc_pad = jnp.zeros((H, H), f32).at[:, :C].set(Wc)
    bc_pad = jnp.zeros((1, H), f32).at[0, :C].set(bc)
    b1r = b1.reshape(1, H)
    b2r = b2.reshape(1, H)
    b3r = b3.reshape(1, H)

    degp = _sc_scatter(ones_table, dst, dst, zeros_hbm)
    m1 = _tc_mm(x_pad, W1)
    dinv, g1 = _tc_prep(degp, m1)

    s1 = _sc_scatter(g1, src, dst, zeros_hbm)
    g2 = _tc_layer(s1, g1, dinv, b1r, W2)
    s2 = _sc_scatter(g2, src, dst, zeros_hbm)
    g3 = _tc_layer(s2, g2, dinv, b2r, W3)
    s3 = _sc_scatter(g3, src, dst, zeros_hbm)
    h_pad, out_pad = _tc_final(s3, g3, dinv, b3r, Wc_pad, bc_pad)

    return out_pad[:N, :C], h_pad[:N]


# speedup vs baseline: 1.8854x; 1.8854x over previous
import functools

import jax
import jax.numpy as jnp
from jax import lax
from jax.experimental import pallas as pl
from jax.experimental.pallas import tpu as pltpu
from jax.experimental.pallas import tpu_sc as plsc

N = 10000
E = 320000
D = 128
H = 128
C = 40

NC = 2
NS = 16
NW = NC * NS

LANES = 128
HP = H // 2
N_PAD = 10112
DUMMY = N_PAD - 1
ROWS_PER_SUB = N_PAD // NS
E_ROWS = 2560
ROWS_PER_W = E_ROWS // NW

NBUF = 2
NPASS = 2
ROWS_PP = ROWS_PER_W // NPASS

_mesh = plsc.VectorSubcoreMesh(core_axis_name="c", subcore_axis_name="s")


@functools.partial(
    pl.kernel,
    out_type=jax.ShapeDtypeStruct((NC, N_PAD, H), jnp.float32),
    mesh=_mesh,
    scratch_types=[
        pltpu.VMEM((ROWS_PER_W, LANES), jnp.int32),
        pltpu.VMEM((LANES, H), jnp.float32),
        pltpu.VMEM_SHARED((N_PAD, H), jnp.float32),
    ],
)
def _sc_degree(dst_hbm, ones_hbm, zeros_hbm, out_hbm, dstv, onesv, acc):
    c = lax.axis_index("c")
    s = lax.axis_index("s")
    wid = c * NS + s
    pltpu.sync_copy(zeros_hbm, acc.at[pl.ds(s * ROWS_PER_SUB, ROWS_PER_SUB)])
    pltpu.sync_copy(dst_hbm.at[pl.ds(wid * ROWS_PER_W, ROWS_PER_W)], dstv)
    pltpu.sync_copy(ones_hbm, onesv)
    plsc.subcore_barrier()

    @pl.loop(0, ROWS_PER_W)
    def _(j):
        pltpu.sync_copy(onesv, acc.at[dstv.at[j]], add=True)

    plsc.subcore_barrier()
    pltpu.sync_copy(
        acc.at[pl.ds(s * ROWS_PER_SUB, ROWS_PER_SUB)],
        out_hbm.at[c, pl.ds(s * ROWS_PER_SUB, ROWS_PER_SUB)],
    )


@functools.partial(
    pl.kernel,
    out_type=jax.ShapeDtypeStruct((NC, N_PAD, H), jnp.float32),
    mesh=_mesh,
    scratch_types=[
        pltpu.VMEM((ROWS_PP, LANES), jnp.int32),
        pltpu.VMEM((ROWS_PP, LANES), jnp.int32),
        pltpu.VMEM((NBUF, LANES, HP), jnp.int32),
        pltpu.VMEM((LANES, H), jnp.float32),
        pltpu.VMEM_SHARED((N_PAD, H), jnp.float32),
    ]
    + [pltpu.SemaphoreType.DMA] * NBUF,
    compiler_params=pltpu.CompilerParams(use_tc_tiling_on_sc=False,
                                         needs_layout_passes=False),
)
def _sc_scatter_packed(gp_hbm, src_hbm, dst_hbm, zeros_hbm, out_hbm,
                       srcv, dstv, rowbuf, fbuf, acc, *gsem):
    c = lax.axis_index("c")
    s = lax.axis_index("s")
    wid = c * NS + s
    pltpu.sync_copy(zeros_hbm, acc.at[pl.ds(s * ROWS_PER_SUB, ROWS_PER_SUB)])
    plsc.subcore_barrier()

    for p in range(NPASS):
        base = wid * ROWS_PER_W + p * ROWS_PP
        pltpu.sync_copy(src_hbm.at[pl.ds(base, ROWS_PP)], srcv)
        pltpu.sync_copy(dst_hbm.at[pl.ds(base, ROWS_PP)], dstv)
        for b in range(NBUF):
            pltpu.async_copy(gp_hbm.at[srcv.at[b]], rowbuf.at[b], gsem[b])

        @pl.loop(0, ROWS_PP, step=NBUF)
        def _(j):
            for b in range(NBUF):
                k = j + b
                pltpu.make_async_copy(
                    gp_hbm.at[srcv.at[k]], rowbuf.at[b], gsem[b]).wait()

                @pl.loop(0, LANES, unroll=8)
                def _(r):
                    for q in range(HP // 16):
                        w = rowbuf[b, r, pl.ds(q * 16, 16)]
                        fbuf[r, pl.ds(q * 16, 16)] = plsc.bitcast(
                            jnp.left_shift(w, 16), jnp.float32)
                        fbuf[r, pl.ds(HP + q * 16, 16)] = plsc.bitcast(
                            jnp.bitwise_and(w, jnp.int32(-65536)), jnp.float32)

                @pl.when(k + NBUF < ROWS_PP)
                def _():
                    pltpu.async_copy(
                        gp_hbm.at[srcv.at[k + NBUF]], rowbuf.at[b], gsem[b])

                pltpu.sync_copy(fbuf, acc.at[dstv.at[k]], add=True)

    plsc.subcore_barrier()
    pltpu.sync_copy(
        acc.at[pl.ds(s * ROWS_PER_SUB, ROWS_PER_SUB)],
        out_hbm.at[c, pl.ds(s * ROWS_PER_SUB, ROWS_PER_SUB)],
    )


BLK = 1264
GRID = N_PAD // BLK


def _tc_mm(x, W):
    def body(x_ref, w_ref, o_ref):
        o_ref[...] = jnp.dot(x_ref[...], w_ref[...],
                             preferred_element_type=jnp.float32)
    return pl.pallas_call(
        body,
        grid=(GRID,),
        in_specs=[pl.BlockSpec((BLK, D), lambda i: (i, 0)),
                  pl.BlockSpec((D, H), lambda i: (0, 0))],
        out_specs=pl.BlockSpec((BLK, H), lambda i: (i, 0)),
        out_shape=jax.ShapeDtypeStruct((N_PAD, H), jnp.float32),
    )(x, W)


def _tc_prep(degp, m1):
    def body(deg_ref, m_ref, dinv_ref, g_ref):
        dinv = lax.rsqrt(deg_ref[0, :, 0:1] + deg_ref[1, :, 0:1] + 1.0)
        dinv_ref[...] = dinv
        g_ref[...] = m_ref[...] * dinv
    return pl.pallas_call(
        body,
        grid=(GRID,),
        in_specs=[pl.BlockSpec((NC, BLK, H), lambda i: (0, i, 0)),
                  pl.BlockSpec((BLK, H), lambda i: (i, 0))],
        out_specs=[pl.BlockSpec((BLK, 1), lambda i: (i, 0)),
                   pl.BlockSpec((BLK, H), lambda i: (i, 0))],
        out_shape=[jax.ShapeDtypeStruct((N_PAD, 1), jnp.float32),
                   jax.ShapeDtypeStruct((N_PAD, H), jnp.float32)],
    )(degp, m1)


def _tc_layer(parts, g_prev, dinv, b, W_next):
    def body(p_ref, g_ref, dinv_ref, b_ref, w_ref, o_ref):
        ssum = p_ref[0] + p_ref[1] + g_ref[...]
        h = jnp.tanh(dinv_ref[...] * ssum + b_ref[...])
        o_ref[...] = jnp.dot(h, w_ref[...],
                             preferred_element_type=jnp.float32) * dinv_ref[...]
    return pl.pallas_call(
        body,
        grid=(GRID,),
        in_specs=[pl.BlockSpec((NC, BLK, H), lambda i: (0, i, 0)),
                  pl.BlockSpec((BLK, H), lambda i: (i, 0)),
                  pl.BlockSpec((BLK, 1), lambda i: (i, 0)),
                  pl.BlockSpec((1, H), lambda i: (0, 0)),
                  pl.BlockSpec((H, H), lambda i: (0, 0))],
        out_specs=pl.BlockSpec((BLK, H), lambda i: (i, 0)),
        out_shape=jax.ShapeDtypeStruct((N_PAD, H), jnp.float32),
    )(parts, g_prev, dinv, b, W_next)


def _tc_final(parts, g_prev, dinv, b, Wc_pad, bc_pad):
    def body(p_ref, g_ref, dinv_ref, b_ref, wc_ref, bc_ref, h_ref, o_ref):
        ssum = p_ref[0] + p_ref[1] + g_ref[...]
        h = jnp.tanh(dinv_ref[...] * ssum + b_ref[...])
        h_ref[...] = h
        o_ref[...] = jnp.dot(h, wc_ref[...],
                             preferred_element_type=jnp.float32) + bc_ref[...]
    return pl.pallas_call(
        body,
        grid=(GRID,),
        in_specs=[pl.BlockSpec((NC, BLK, H), lambda i: (0, i, 0)),
                  pl.BlockSpec((BLK, H), lambda i: (i, 0)),
                  pl.BlockSpec((BLK, 1), lambda i: (i, 0)),
                  pl.BlockSpec((1, H), lambda i: (0, 0)),
                  pl.BlockSpec((H, H), lambda i: (0, 0)),
                  pl.BlockSpec((1, H), lambda i: (0, 0))],
        out_specs=[pl.BlockSpec((BLK, H), lambda i: (i, 0)),
                   pl.BlockSpec((BLK, H), lambda i: (i, 0))],
        out_shape=[jax.ShapeDtypeStruct((N_PAD, H), jnp.float32),
                   jax.ShapeDtypeStruct((N_PAD, H), jnp.float32)],
    )(parts, g_prev, dinv, b, Wc_pad, bc_pad)


def _pack(g):
    gb = g.astype(jnp.bfloat16)
    return jax.lax.bitcast_convert_type(
        jnp.stack([gb[:, :HP], gb[:, HP:]], axis=-1), jnp.int32)


@jax.jit
def kernel(x, edge_index, W1, b1, W2, b2, W3, b3, Wc, bc):
    f32 = jnp.float32
    x_pad = jnp.zeros((N_PAD, D), f32).at[:N].set(x)

    e_pad = E_ROWS * LANES - E
    src = jnp.concatenate([edge_index[0], jnp.zeros((e_pad,), jnp.int32)])
    dst = jnp.concatenate(
        [edge_index[1], jnp.full((e_pad,), DUMMY, jnp.int32)])
    src = src.reshape(E_ROWS, LANES)
    dst = dst.reshape(E_ROWS, LANES)

    ones_hbm = jnp.ones((LANES, H), f32)
    zeros_hbm = jnp.zeros((ROWS_PER_SUB, H), f32)

    Wc_pad = jnp.zeros((H, H), f32).at[:, :C].set(Wc)
    bc_pad = jnp.zeros((1, H), f32).at[0, :C].set(bc)
    b1r = b1.reshape(1, H)
    b2r = b2.reshape(1, H)
    b3r = b3.reshape(1, H)

    degp = _sc_degree(dst, ones_hbm, zeros_hbm)
    m1 = _tc_mm(x_pad, W1)
    dinv, g1 = _tc_prep(degp, m1)

    s1 = _sc_scatter_packed(_pack(g1), src, dst, zeros_hbm)
    g2 = _tc_layer(s1, g1, dinv, b1r, W2)
    s2 = _sc_scatter_packed(_pack(g2), src, dst, zeros_hbm)
    g3 = _tc_layer(s2, g2, dinv, b2r, W3)
    s3 = _sc_scatter_packed(_pack(g3), src, dst, zeros_hbm)
    h_pad, out_pad = _tc_final(s3, g3, dinv, b3r, Wc_pad, bc_pad)

    return out_pad[:N, :C], h_pad[:N]


# speedup vs baseline: 1.9696x; 1.0447x over previous
import functools

import jax
import jax.numpy as jnp
from jax import lax
from jax.experimental import pallas as pl
from jax.experimental.pallas import tpu as pltpu
from jax.experimental.pallas import tpu_sc as plsc

N = 10000
E = 320000
D = 128
H = 128
C = 40

NC = 2
NS = 16
NW = NC * NS

LANES = 128
HP = H // 2
N_PAD = 10112
DUMMY = N_PAD - 1
ROWS_PER_SUB = N_PAD // NS
E_ROWS = 2560
ROWS_PER_W = E_ROWS // NW

NBUF = 2
NPASS = 2
ROWS_PP = ROWS_PER_W // NPASS

_mesh = plsc.VectorSubcoreMesh(core_axis_name="c", subcore_axis_name="s")


@functools.partial(
    pl.kernel,
    out_type=jax.ShapeDtypeStruct((NC, N_PAD, H), jnp.float32),
    mesh=_mesh,
    scratch_types=[
        pltpu.VMEM((ROWS_PER_W, LANES), jnp.int32),
        pltpu.VMEM((LANES, H), jnp.float32),
        pltpu.VMEM_SHARED((N_PAD, H), jnp.float32),
        pltpu.SemaphoreType.DMA,
    ],
)
def _sc_degree(dst_hbm, ones_hbm, zeros_hbm, out_hbm, dstv, onesv, acc, sem):
    c = lax.axis_index("c")
    s = lax.axis_index("s")
    wid = c * NS + s
    pltpu.sync_copy(zeros_hbm, acc.at[pl.ds(s * ROWS_PER_SUB, ROWS_PER_SUB)])
    pltpu.sync_copy(dst_hbm.at[pl.ds(wid * ROWS_PER_W, ROWS_PER_W)], dstv)
    pltpu.sync_copy(ones_hbm, onesv)
    plsc.subcore_barrier()

    @pl.loop(0, ROWS_PER_W)
    def _(j):
        pltpu.async_copy(onesv, acc.at[dstv.at[j]], sem, add=True)

    @pl.loop(0, ROWS_PER_W)
    def _(j):
        pltpu.make_async_copy(onesv, acc.at[dstv.at[j]], sem).wait()

    plsc.subcore_barrier()
    pltpu.sync_copy(
        acc.at[pl.ds(s * ROWS_PER_SUB, ROWS_PER_SUB)],
        out_hbm.at[c, pl.ds(s * ROWS_PER_SUB, ROWS_PER_SUB)],
    )


@functools.partial(
    pl.kernel,
    out_type=jax.ShapeDtypeStruct((NC, N_PAD, H), jnp.float32),
    mesh=_mesh,
    scratch_types=[
        pltpu.VMEM((ROWS_PP, LANES), jnp.int32),
        pltpu.VMEM((ROWS_PP, LANES), jnp.int32),
        pltpu.VMEM((NBUF, LANES, HP), jnp.int32),
        pltpu.VMEM((LANES, H), jnp.float32),
        pltpu.VMEM_SHARED((N_PAD, H), jnp.float32),
    ]
    + [pltpu.SemaphoreType.DMA] * NBUF,
    compiler_params=pltpu.CompilerParams(use_tc_tiling_on_sc=False,
                                         needs_layout_passes=False),
)
def _sc_scatter_packed(gp_hbm, src_hbm, dst_hbm, zeros_hbm, out_hbm,
                       srcv, dstv, rowbuf, fbuf, acc, *gsem):
    c = lax.axis_index("c")
    s = lax.axis_index("s")
    wid = c * NS + s
    pltpu.sync_copy(zeros_hbm, acc.at[pl.ds(s * ROWS_PER_SUB, ROWS_PER_SUB)])
    plsc.subcore_barrier()

    for p in range(NPASS):
        base = wid * ROWS_PER_W + p * ROWS_PP
        pltpu.sync_copy(src_hbm.at[pl.ds(base, ROWS_PP)], srcv)
        pltpu.sync_copy(dst_hbm.at[pl.ds(base, ROWS_PP)], dstv)
        for b in range(NBUF):
            pltpu.async_copy(gp_hbm.at[srcv.at[b]], rowbuf.at[b], gsem[b])

        @pl.loop(0, ROWS_PP, step=NBUF)
        def _(j):
            for b in range(NBUF):
                k = j + b
                pltpu.make_async_copy(
                    gp_hbm.at[srcv.at[k]], rowbuf.at[b], gsem[b]).wait()

                @pl.loop(0, LANES, unroll=8)
                def _(r):
                    for q in range(HP // 16):
                        w = rowbuf[b, r, pl.ds(q * 16, 16)]
                        fbuf[r, pl.ds(q * 16, 16)] = plsc.bitcast(
                            jnp.left_shift(w, 16), jnp.float32)
                        fbuf[r, pl.ds(HP + q * 16, 16)] = plsc.bitcast(
                            jnp.bitwise_and(w, jnp.int32(-65536)), jnp.float32)

                @pl.when(k + NBUF < ROWS_PP)
                def _():
                    pltpu.async_copy(
                        gp_hbm.at[srcv.at[k + NBUF]], rowbuf.at[b], gsem[b])

                pltpu.sync_copy(fbuf, acc.at[dstv.at[k]], add=True)

    plsc.subcore_barrier()
    pltpu.sync_copy(
        acc.at[pl.ds(s * ROWS_PER_SUB, ROWS_PER_SUB)],
        out_hbm.at[c, pl.ds(s * ROWS_PER_SUB, ROWS_PER_SUB)],
    )


BLK = 1264
GRID = N_PAD // BLK


def _tc_mm(x, W):
    def body(x_ref, w_ref, o_ref):
        o_ref[...] = jnp.dot(x_ref[...], w_ref[...],
                             preferred_element_type=jnp.float32)
    return pl.pallas_call(
        body,
        grid=(GRID,),
        in_specs=[pl.BlockSpec((BLK, D), lambda i: (i, 0)),
                  pl.BlockSpec((D, H), lambda i: (0, 0))],
        out_specs=pl.BlockSpec((BLK, H), lambda i: (i, 0)),
        out_shape=jax.ShapeDtypeStruct((N_PAD, H), jnp.float32),
    )(x, W)


def _tc_prep(degp, m1):
    def body(deg_ref, m_ref, dinv_ref, g_ref):
        dinv = lax.rsqrt(deg_ref[0, :, 0:1] + deg_ref[1, :, 0:1] + 1.0)
        dinv_ref[...] = dinv
        g_ref[...] = m_ref[...] * dinv
    return pl.pallas_call(
        body,
        grid=(GRID,),
        in_specs=[pl.BlockSpec((NC, BLK, H), lambda i: (0, i, 0)),
                  pl.BlockSpec((BLK, H), lambda i: (i, 0))],
        out_specs=[pl.BlockSpec((BLK, 1), lambda i: (i, 0)),
                   pl.BlockSpec((BLK, H), lambda i: (i, 0))],
        out_shape=[jax.ShapeDtypeStruct((N_PAD, 1), jnp.float32),
                   jax.ShapeDtypeStruct((N_PAD, H), jnp.float32)],
    )(degp, m1)


def _tc_layer(parts, g_prev, dinv, b, W_next):
    def body(p_ref, g_ref, dinv_ref, b_ref, w_ref, o_ref):
        ssum = p_ref[0] + p_ref[1] + g_ref[...]
        h = jnp.tanh(dinv_ref[...] * ssum + b_ref[...])
        o_ref[...] = jnp.dot(h, w_ref[...],
                             preferred_element_type=jnp.float32) * dinv_ref[...]
    return pl.pallas_call(
        body,
        grid=(GRID,),
        in_specs=[pl.BlockSpec((NC, BLK, H), lambda i: (0, i, 0)),
                  pl.BlockSpec((BLK, H), lambda i: (i, 0)),
                  pl.BlockSpec((BLK, 1), lambda i: (i, 0)),
                  pl.BlockSpec((1, H), lambda i: (0, 0)),
                  pl.BlockSpec((H, H), lambda i: (0, 0))],
        out_specs=pl.BlockSpec((BLK, H), lambda i: (i, 0)),
        out_shape=jax.ShapeDtypeStruct((N_PAD, H), jnp.float32),
    )(parts, g_prev, dinv, b, W_next)


def _tc_final(parts, g_prev, dinv, b, Wc_pad, bc_pad):
    def body(p_ref, g_ref, dinv_ref, b_ref, wc_ref, bc_ref, h_ref, o_ref):
        ssum = p_ref[0] + p_ref[1] + g_ref[...]
        h = jnp.tanh(dinv_ref[...] * ssum + b_ref[...])
        h_ref[...] = h
        o_ref[...] = jnp.dot(h, wc_ref[...],
                             preferred_element_type=jnp.float32) + bc_ref[...]
    return pl.pallas_call(
        body,
        grid=(GRID,),
        in_specs=[pl.BlockSpec((NC, BLK, H), lambda i: (0, i, 0)),
                  pl.BlockSpec((BLK, H), lambda i: (i, 0)),
                  pl.BlockSpec((BLK, 1), lambda i: (i, 0)),
                  pl.BlockSpec((1, H), lambda i: (0, 0)),
                  pl.BlockSpec((H, H), lambda i: (0, 0)),
                  pl.BlockSpec((1, H), lambda i: (0, 0))],
        out_specs=[pl.BlockSpec((BLK, H), lambda i: (i, 0)),
                   pl.BlockSpec((BLK, H), lambda i: (i, 0))],
        out_shape=[jax.ShapeDtypeStruct((N_PAD, H), jnp.float32),
                   jax.ShapeDtypeStruct((N_PAD, H), jnp.float32)],
    )(parts, g_prev, dinv, b, Wc_pad, bc_pad)


def _pack(g):
    gb = g.astype(jnp.bfloat16)
    return jax.lax.bitcast_convert_type(
        jnp.stack([gb[:, :HP], gb[:, HP:]], axis=-1), jnp.int32)


@jax.jit
def kernel(x, edge_index, W1, b1, W2, b2, W3, b3, Wc, bc):
    f32 = jnp.float32
    x_pad = jnp.zeros((N_PAD, D), f32).at[:N].set(x)

    e_pad = E_ROWS * LANES - E
    src = jnp.concatenate([edge_index[0], jnp.zeros((e_pad,), jnp.int32)])
    dst = jnp.concatenate(
        [edge_index[1], jnp.full((e_pad,), DUMMY, jnp.int32)])
    src = src.reshape(E_ROWS, LANES)
    dst = dst.reshape(E_ROWS, LANES)

    ones_hbm = jnp.ones((LANES, H), f32)
    zeros_hbm = jnp.zeros((ROWS_PER_SUB, H), f32)

    Wc_pad = jnp.zeros((H, H), f32).at[:, :C].set(Wc)
    bc_pad = jnp.zeros((1, H), f32).at[0, :C].set(bc)
    b1r = b1.reshape(1, H)
    b2r = b2.reshape(1, H)
    b3r = b3.reshape(1, H)

    degp = _sc_degree(dst, ones_hbm, zeros_hbm)
    m1 = _tc_mm(x_pad, W1)
    dinv, g1 = _tc_prep(degp, m1)

    s1 = _sc_scatter_packed(_pack(g1), src, dst, zeros_hbm)
    g2 = _tc_layer(s1, g1, dinv, b1r, W2)
    s2 = _sc_scatter_packed(_pack(g2), src, dst, zeros_hbm)
    g3 = _tc_layer(s2, g2, dinv, b2r, W3)
    s3 = _sc_scatter_packed(_pack(g3), src, dst, zeros_hbm)
    h_pad, out_pad = _tc_final(s3, g3, dinv, b3r, Wc_pad, bc_pad)

    return out_pad[:N, :C], h_pad[:N]


# speedup vs baseline: 1.9978x; 1.0143x over previous
import functools

import jax
import jax.numpy as jnp
from jax import lax
from jax.experimental import pallas as pl
from jax.experimental.pallas import tpu as pltpu
from jax.experimental.pallas import tpu_sc as plsc

N = 10000
E = 320000
D = 128
H = 128
C = 40

NC = 2
NS = 16
NW = NC * NS

LANES = 128
HP = H // 2
N_PAD = 10112
DUMMY = N_PAD - 1
ROWS_PER_SUB = N_PAD // NS
E_ROWS = 2560
ROWS_PER_W = E_ROWS // NW

NBUF = 2
NPASS = 2
ROWS_PP = ROWS_PER_W // NPASS

_mesh = plsc.VectorSubcoreMesh(core_axis_name="c", subcore_axis_name="s")


DW = 16


@functools.partial(
    pl.kernel,
    out_type=jax.ShapeDtypeStruct((NC, N_PAD, DW), jnp.float32),
    mesh=_mesh,
    scratch_types=[
        pltpu.VMEM((ROWS_PER_W, LANES), jnp.int32),
        pltpu.VMEM((LANES, DW), jnp.float32),
        pltpu.VMEM_SHARED((N_PAD, DW), jnp.float32),
        pltpu.SemaphoreType.DMA,
    ],
    compiler_params=pltpu.CompilerParams(use_tc_tiling_on_sc=False,
                                         needs_layout_passes=False),
)
def _sc_degree(dst_hbm, ones_hbm, zeros_hbm, out_hbm, dstv, onesv, acc, sem):
    c = lax.axis_index("c")
    s = lax.axis_index("s")
    wid = c * NS + s
    pltpu.sync_copy(zeros_hbm, acc.at[pl.ds(s * ROWS_PER_SUB, ROWS_PER_SUB)])
    pltpu.sync_copy(dst_hbm.at[pl.ds(wid * ROWS_PER_W, ROWS_PER_W)], dstv)
    pltpu.sync_copy(ones_hbm, onesv)
    plsc.subcore_barrier()

    @pl.loop(0, ROWS_PER_W)
    def _(j):
        pltpu.async_copy(onesv, acc.at[dstv.at[j]], sem, add=True)

    @pl.loop(0, ROWS_PER_W)
    def _(j):
        pltpu.make_async_copy(onesv, acc.at[dstv.at[j]], sem).wait()

    plsc.subcore_barrier()
    pltpu.sync_copy(
        acc.at[pl.ds(s * ROWS_PER_SUB, ROWS_PER_SUB)],
        out_hbm.at[c, pl.ds(s * ROWS_PER_SUB, ROWS_PER_SUB)],
    )


@functools.partial(
    pl.kernel,
    out_type=jax.ShapeDtypeStruct((NC, N_PAD, H), jnp.float32),
    mesh=_mesh,
    scratch_types=[
        pltpu.VMEM((ROWS_PP, LANES), jnp.int32),
        pltpu.VMEM((ROWS_PP, LANES), jnp.int32),
        pltpu.VMEM((NBUF, LANES, HP), jnp.int32),
        pltpu.VMEM((LANES, H), jnp.float32),
        pltpu.VMEM_SHARED((N_PAD, H), jnp.float32),
    ]
    + [pltpu.SemaphoreType.DMA] * NBUF,
    compiler_params=pltpu.CompilerParams(use_tc_tiling_on_sc=False,
                                         needs_layout_passes=False),
)
def _sc_scatter_packed(gp_hbm, src_hbm, dst_hbm, zeros_hbm, out_hbm,
                       srcv, dstv, rowbuf, fbuf, acc, *gsem):
    c = lax.axis_index("c")
    s = lax.axis_index("s")
    wid = c * NS + s
    pltpu.sync_copy(zeros_hbm, acc.at[pl.ds(s * ROWS_PER_SUB, ROWS_PER_SUB)])
    plsc.subcore_barrier()

    for p in range(NPASS):
        base = wid * ROWS_PER_W + p * ROWS_PP
        pltpu.sync_copy(src_hbm.at[pl.ds(base, ROWS_PP)], srcv)
        pltpu.sync_copy(dst_hbm.at[pl.ds(base, ROWS_PP)], dstv)
        for b in range(NBUF):
            pltpu.async_copy(gp_hbm.at[srcv.at[b]], rowbuf.at[b], gsem[b])

        @pl.loop(0, ROWS_PP, step=NBUF)
        def _(j):
            for b in range(NBUF):
                k = j + b
                pltpu.make_async_copy(
                    gp_hbm.at[srcv.at[k]], rowbuf.at[b], gsem[b]).wait()

                @pl.loop(0, LANES, unroll=8)
                def _(r):
                    for q in range(HP // 16):
                        w = rowbuf[b, r, pl.ds(q * 16, 16)]
                        fbuf[r, pl.ds(q * 16, 16)] = plsc.bitcast(
                            jnp.left_shift(w, 16), jnp.float32)
                        fbuf[r, pl.ds(HP + q * 16, 16)] = plsc.bitcast(
                            jnp.bitwise_and(w, jnp.int32(-65536)), jnp.float32)

                @pl.when(k + NBUF < ROWS_PP)
                def _():
                    pltpu.async_copy(
                        gp_hbm.at[srcv.at[k + NBUF]], rowbuf.at[b], gsem[b])

                pltpu.sync_copy(fbuf, acc.at[dstv.at[k]], add=True)

    plsc.subcore_barrier()
    pltpu.sync_copy(
        acc.at[pl.ds(s * ROWS_PER_SUB, ROWS_PER_SUB)],
        out_hbm.at[c, pl.ds(s * ROWS_PER_SUB, ROWS_PER_SUB)],
    )


BLK = 1264
GRID = N_PAD // BLK


def _tc_mm(x, W):
    def body(x_ref, w_ref, o_ref):
        o_ref[...] = jnp.dot(x_ref[...], w_ref[...],
                             preferred_element_type=jnp.float32)
    return pl.pallas_call(
        body,
        grid=(GRID,),
        in_specs=[pl.BlockSpec((BLK, D), lambda i: (i, 0)),
                  pl.BlockSpec((D, H), lambda i: (0, 0))],
        out_specs=pl.BlockSpec((BLK, H), lambda i: (i, 0)),
        out_shape=jax.ShapeDtypeStruct((N_PAD, H), jnp.float32),
    )(x, W)


def _tc_prep(degp, m1):
    def body(deg_ref, m_ref, dinv_ref, g_ref):
        dinv = lax.rsqrt(deg_ref[0, :, 0:1] + deg_ref[1, :, 0:1] + 1.0)
        dinv_ref[...] = dinv
        g_ref[...] = m_ref[...] * dinv
    return pl.pallas_call(
        body,
        grid=(GRID,),
        in_specs=[pl.BlockSpec((NC, BLK, DW), lambda i: (0, i, 0)),
                  pl.BlockSpec((BLK, H), lambda i: (i, 0))],
        out_specs=[pl.BlockSpec((BLK, 1), lambda i: (i, 0)),
                   pl.BlockSpec((BLK, H), lambda i: (i, 0))],
        out_shape=[jax.ShapeDtypeStruct((N_PAD, 1), jnp.float32),
                   jax.ShapeDtypeStruct((N_PAD, H), jnp.float32)],
    )(degp, m1)


def _tc_layer(parts, g_prev, dinv, b, W_next):
    def body(p_ref, g_ref, dinv_ref, b_ref, w_ref, o_ref):
        ssum = p_ref[0] + p_ref[1] + g_ref[...]
        h = jnp.tanh(dinv_ref[...] * ssum + b_ref[...])
        o_ref[...] = jnp.dot(h, w_ref[...],
                             preferred_element_type=jnp.float32) * dinv_ref[...]
    return pl.pallas_call(
        body,
        grid=(GRID,),
        in_specs=[pl.BlockSpec((NC, BLK, H), lambda i: (0, i, 0)),
                  pl.BlockSpec((BLK, H), lambda i: (i, 0)),
                  pl.BlockSpec((BLK, 1), lambda i: (i, 0)),
                  pl.BlockSpec((1, H), lambda i: (0, 0)),
                  pl.BlockSpec((H, H), lambda i: (0, 0))],
        out_specs=pl.BlockSpec((BLK, H), lambda i: (i, 0)),
        out_shape=jax.ShapeDtypeStruct((N_PAD, H), jnp.float32),
    )(parts, g_prev, dinv, b, W_next)


def _tc_final(parts, g_prev, dinv, b, Wc_pad, bc_pad):
    def body(p_ref, g_ref, dinv_ref, b_ref, wc_ref, bc_ref, h_ref, o_ref):
        ssum = p_ref[0] + p_ref[1] + g_ref[...]
        h = jnp.tanh(dinv_ref[...] * ssum + b_ref[...])
        h_ref[...] = h
        o_ref[...] = jnp.dot(h, wc_ref[...],
                             preferred_element_type=jnp.float32) + bc_ref[...]
    return pl.pallas_call(
        body,
        grid=(GRID,),
        in_specs=[pl.BlockSpec((NC, BLK, H), lambda i: (0, i, 0)),
                  pl.BlockSpec((BLK, H), lambda i: (i, 0)),
                  pl.BlockSpec((BLK, 1), lambda i: (i, 0)),
                  pl.BlockSpec((1, H), lambda i: (0, 0)),
                  pl.BlockSpec((H, H), lambda i: (0, 0)),
                  pl.BlockSpec((1, H), lambda i: (0, 0))],
        out_specs=[pl.BlockSpec((BLK, H), lambda i: (i, 0)),
                   pl.BlockSpec((BLK, H), lambda i: (i, 0))],
        out_shape=[jax.ShapeDtypeStruct((N_PAD, H), jnp.float32),
                   jax.ShapeDtypeStruct((N_PAD, H), jnp.float32)],
    )(parts, g_prev, dinv, b, Wc_pad, bc_pad)


def _pack(g):
    gb = g.astype(jnp.bfloat16)
    return jax.lax.bitcast_convert_type(
        jnp.stack([gb[:, :HP], gb[:, HP:]], axis=-1), jnp.int32)


@jax.jit
def kernel(x, edge_index, W1, b1, W2, b2, W3, b3, Wc, bc):
    f32 = jnp.float32
    x_pad = jnp.zeros((N_PAD, D), f32).at[:N].set(x)

    e_pad = E_ROWS * LANES - E
    src = jnp.concatenate([edge_index[0], jnp.zeros((e_pad,), jnp.int32)])
    dst = jnp.concatenate(
        [edge_index[1], jnp.full((e_pad,), DUMMY, jnp.int32)])
    src = src.reshape(E_ROWS, LANES)
    dst = dst.reshape(E_ROWS, LANES)

    ones_hbm = jnp.ones((LANES, DW), f32)
    zeros_dw = jnp.zeros((ROWS_PER_SUB, DW), f32)
    zeros_hbm = jnp.zeros((ROWS_PER_SUB, H), f32)

    Wc_pad = jnp.zeros((H, H), f32).at[:, :C].set(Wc)
    bc_pad = jnp.zeros((1, H), f32).at[0, :C].set(bc)
    b1r = b1.reshape(1, H)
    b2r = b2.reshape(1, H)
    b3r = b3.reshape(1, H)

    degp = _sc_degree(dst, ones_hbm, zeros_dw)
    m1 = _tc_mm(x_pad, W1)
    dinv, g1 = _tc_prep(degp, m1)

    s1 = _sc_scatter_packed(_pack(g1), src, dst, zeros_hbm)
    g2 = _tc_layer(s1, g1, dinv, b1r, W2)
    s2 = _sc_scatter_packed(_pack(g2), src, dst, zeros_hbm)
    g3 = _tc_layer(s2, g2, dinv, b2r, W3)
    s3 = _sc_scatter_packed(_pack(g3), src, dst, zeros_hbm)
    h_pad, out_pad = _tc_final(s3, g3, dinv, b3r, Wc_pad, bc_pad)

    return out_pad[:N, :C], h_pad[:N]


# speedup vs baseline: 2.1032x; 1.0528x over previous
import functools

import jax
import jax.numpy as jnp
from jax import lax
from jax.experimental import pallas as pl
from jax.experimental.pallas import tpu as pltpu
from jax.experimental.pallas import tpu_sc as plsc

N = 10000
E = 320000
D = 128
H = 128
C = 40

NC = 2
NS = 16
NW = NC * NS

LANES = 128
HP = H // 2
N_PAD = 10112
DUMMY = N_PAD - 1
ROWS_PER_SUB = N_PAD // NS
E_ROWS = 2560
ROWS_PER_W = E_ROWS // NW

NBUF = 2
NPASS = 2
ROWS_PP = ROWS_PER_W // NPASS

_mesh = plsc.VectorSubcoreMesh(core_axis_name="c", subcore_axis_name="s")


DW = 16


@functools.partial(
    pl.kernel,
    out_type=jax.ShapeDtypeStruct((NC, N_PAD, DW), jnp.float32),
    mesh=_mesh,
    scratch_types=[
        pltpu.VMEM((ROWS_PER_W, LANES), jnp.int32),
        pltpu.VMEM((LANES, DW), jnp.float32),
        pltpu.VMEM_SHARED((N_PAD, DW), jnp.float32),
        pltpu.SemaphoreType.DMA,
    ],
    compiler_params=pltpu.CompilerParams(use_tc_tiling_on_sc=False,
                                         needs_layout_passes=False),
)
def _sc_degree(dst_hbm, ones_hbm, zeros_hbm, out_hbm, dstv, onesv, acc, sem):
    c = lax.axis_index("c")
    s = lax.axis_index("s")
    wid = c * NS + s
    pltpu.sync_copy(zeros_hbm, acc.at[pl.ds(s * ROWS_PER_SUB, ROWS_PER_SUB)])
    pltpu.sync_copy(dst_hbm.at[pl.ds(wid * ROWS_PER_W, ROWS_PER_W)], dstv)
    pltpu.sync_copy(ones_hbm, onesv)
    plsc.subcore_barrier()

    @pl.loop(0, ROWS_PER_W)
    def _(j):
        pltpu.async_copy(onesv, acc.at[dstv.at[j]], sem, add=True)

    @pl.loop(0, ROWS_PER_W)
    def _(j):
        pltpu.make_async_copy(onesv, acc.at[dstv.at[j]], sem).wait()

    plsc.subcore_barrier()
    pltpu.sync_copy(
        acc.at[pl.ds(s * ROWS_PER_SUB, ROWS_PER_SUB)],
        out_hbm.at[c, pl.ds(s * ROWS_PER_SUB, ROWS_PER_SUB)],
    )


@functools.partial(
    pl.kernel,
    out_type=jax.ShapeDtypeStruct((NC, N_PAD, H), jnp.float32),
    mesh=_mesh,
    scratch_types=[
        pltpu.VMEM((ROWS_PP, LANES), jnp.int32),
        pltpu.VMEM((ROWS_PP, LANES), jnp.int32),
        pltpu.VMEM((NBUF, LANES, HP), jnp.int32),
        pltpu.VMEM((LANES, H), jnp.float32),
        pltpu.VMEM_SHARED((N_PAD, H), jnp.float32),
    ]
    + [pltpu.SemaphoreType.DMA] * NBUF,
    compiler_params=pltpu.CompilerParams(use_tc_tiling_on_sc=False,
                                         needs_layout_passes=False),
)
def _sc_scatter_packed(gp_hbm, src_hbm, dst_hbm, zeros_hbm, out_hbm,
                       srcv, dstv, rowbuf, fbuf, acc, *gsem):
    c = lax.axis_index("c")
    s = lax.axis_index("s")
    wid = c * NS + s
    pltpu.sync_copy(zeros_hbm, acc.at[pl.ds(s * ROWS_PER_SUB, ROWS_PER_SUB)])
    plsc.subcore_barrier()

    for p in range(NPASS):
        base = wid * ROWS_PER_W + p * ROWS_PP
        pltpu.sync_copy(src_hbm.at[pl.ds(base, ROWS_PP)], srcv)
        pltpu.sync_copy(dst_hbm.at[pl.ds(base, ROWS_PP)], dstv)
        for b in range(NBUF):
            pltpu.async_copy(gp_hbm.at[srcv.at[b]], rowbuf.at[b], gsem[b])

        @pl.loop(0, ROWS_PP, step=NBUF)
        def _(j):
            for b in range(NBUF):
                k = j + b
                pltpu.make_async_copy(
                    gp_hbm.at[srcv.at[k]], rowbuf.at[b], gsem[b]).wait()

                @pl.loop(0, LANES, unroll=8)
                def _(r):
                    for q in range(HP // 16):
                        w = rowbuf[b, r, pl.ds(q * 16, 16)]
                        fbuf[r, pl.ds(q * 16, 16)] = plsc.bitcast(
                            jnp.left_shift(w, 16), jnp.float32)
                        fbuf[r, pl.ds(HP + q * 16, 16)] = plsc.bitcast(
                            jnp.bitwise_and(w, jnp.int32(-65536)), jnp.float32)

                @pl.when(k + NBUF < ROWS_PP)
                def _():
                    pltpu.async_copy(
                        gp_hbm.at[srcv.at[k + NBUF]], rowbuf.at[b], gsem[b])

                pltpu.sync_copy(fbuf, acc.at[dstv.at[k]], add=True)

    plsc.subcore_barrier()
    pltpu.sync_copy(
        acc.at[pl.ds(s * ROWS_PER_SUB, ROWS_PER_SUB)],
        out_hbm.at[c, pl.ds(s * ROWS_PER_SUB, ROWS_PER_SUB)],
    )


BLK = 1264
GRID = N_PAD // BLK


def _tc_mm(x, W):
    def body(x_ref, w_ref, o_ref):
        o_ref[...] = jnp.dot(x_ref[...], w_ref[...],
                             preferred_element_type=jnp.float32)
    return pl.pallas_call(
        body,
        grid=(GRID,),
        in_specs=[pl.BlockSpec((BLK, D), lambda i: (i, 0)),
                  pl.BlockSpec((D, H), lambda i: (0, 0))],
        out_specs=pl.BlockSpec((BLK, H), lambda i: (i, 0)),
        out_shape=jax.ShapeDtypeStruct((N_PAD, H), jnp.float32),
    )(x, W)


def _pack_block(g):
    gb = g.astype(jnp.bfloat16)
    lo = lax.bitcast_convert_type(gb[:, :HP], jnp.uint16).astype(jnp.int32)
    hi = lax.bitcast_convert_type(gb[:, HP:], jnp.uint16).astype(jnp.int32)
    return jnp.bitwise_or(lo, jnp.left_shift(hi, 16))


def _tc_prep(degp, m1):
    def body(deg_ref, m_ref, dinv_ref, g_ref, gp_ref):
        dinv = lax.rsqrt(deg_ref[0, :, 0:1] + deg_ref[1, :, 0:1] + 1.0)
        dinv_ref[...] = dinv
        g = m_ref[...] * dinv
        g_ref[...] = g
        gp_ref[...] = _pack_block(g)
    return pl.pallas_call(
        body,
        grid=(GRID,),
        in_specs=[pl.BlockSpec((NC, BLK, DW), lambda i: (0, i, 0)),
                  pl.BlockSpec((BLK, H), lambda i: (i, 0))],
        out_specs=[pl.BlockSpec((BLK, 1), lambda i: (i, 0)),
                   pl.BlockSpec((BLK, H), lambda i: (i, 0)),
                   pl.BlockSpec((BLK, HP), lambda i: (i, 0))],
        out_shape=[jax.ShapeDtypeStruct((N_PAD, 1), jnp.float32),
                   jax.ShapeDtypeStruct((N_PAD, H), jnp.float32),
                   jax.ShapeDtypeStruct((N_PAD, HP), jnp.int32)],
    )(degp, m1)


def _tc_layer(parts, g_prev, dinv, b, W_next):
    def body(p_ref, g_ref, dinv_ref, b_ref, w_ref, o_ref, gp_ref):
        ssum = p_ref[0] + p_ref[1] + g_ref[...]
        h = jnp.tanh(dinv_ref[...] * ssum + b_ref[...])
        g = jnp.dot(h, w_ref[...],
                    preferred_element_type=jnp.float32) * dinv_ref[...]
        o_ref[...] = g
        gp_ref[...] = _pack_block(g)
    return pl.pallas_call(
        body,
        grid=(GRID,),
        in_specs=[pl.BlockSpec((NC, BLK, H), lambda i: (0, i, 0)),
                  pl.BlockSpec((BLK, H), lambda i: (i, 0)),
                  pl.BlockSpec((BLK, 1), lambda i: (i, 0)),
                  pl.BlockSpec((1, H), lambda i: (0, 0)),
                  pl.BlockSpec((H, H), lambda i: (0, 0))],
        out_specs=[pl.BlockSpec((BLK, H), lambda i: (i, 0)),
                   pl.BlockSpec((BLK, HP), lambda i: (i, 0))],
        out_shape=[jax.ShapeDtypeStruct((N_PAD, H), jnp.float32),
                   jax.ShapeDtypeStruct((N_PAD, HP), jnp.int32)],
    )(parts, g_prev, dinv, b, W_next)


def _tc_final(parts, g_prev, dinv, b, Wc_pad, bc_pad):
    def body(p_ref, g_ref, dinv_ref, b_ref, wc_ref, bc_ref, h_ref, o_ref):
        ssum = p_ref[0] + p_ref[1] + g_ref[...]
        h = jnp.tanh(dinv_ref[...] * ssum + b_ref[...])
        h_ref[...] = h
        o_ref[...] = jnp.dot(h, wc_ref[...],
                             preferred_element_type=jnp.float32) + bc_ref[...]
    return pl.pallas_call(
        body,
        grid=(GRID,),
        in_specs=[pl.BlockSpec((NC, BLK, H), lambda i: (0, i, 0)),
                  pl.BlockSpec((BLK, H), lambda i: (i, 0)),
                  pl.BlockSpec((BLK, 1), lambda i: (i, 0)),
                  pl.BlockSpec((1, H), lambda i: (0, 0)),
                  pl.BlockSpec((H, H), lambda i: (0, 0)),
                  pl.BlockSpec((1, H), lambda i: (0, 0))],
        out_specs=[pl.BlockSpec((BLK, H), lambda i: (i, 0)),
                   pl.BlockSpec((BLK, H), lambda i: (i, 0))],
        out_shape=[jax.ShapeDtypeStruct((N_PAD, H), jnp.float32),
                   jax.ShapeDtypeStruct((N_PAD, H), jnp.float32)],
    )(parts, g_prev, dinv, b, Wc_pad, bc_pad)


@jax.jit
def kernel(x, edge_index, W1, b1, W2, b2, W3, b3, Wc, bc):
    f32 = jnp.float32
    x_pad = jnp.zeros((N_PAD, D), f32).at[:N].set(x)

    e_pad = E_ROWS * LANES - E
    src = jnp.concatenate([edge_index[0], jnp.zeros((e_pad,), jnp.int32)])
    dst = jnp.concatenate(
        [edge_index[1], jnp.full((e_pad,), DUMMY, jnp.int32)])
    src = src.reshape(E_ROWS, LANES)
    dst = dst.reshape(E_ROWS, LANES)

    ones_hbm = jnp.ones((LANES, DW), f32)
    zeros_dw = jnp.zeros((ROWS_PER_SUB, DW), f32)
    zeros_hbm = jnp.zeros((ROWS_PER_SUB, H), f32)

    Wc_pad = jnp.zeros((H, H), f32).at[:, :C].set(Wc)
    bc_pad = jnp.zeros((1, H), f32).at[0, :C].set(bc)
    b1r = b1.reshape(1, H)
    b2r = b2.reshape(1, H)
    b3r = b3.reshape(1, H)

    degp = _sc_degree(dst, ones_hbm, zeros_dw)
    m1 = _tc_mm(x_pad, W1)
    dinv, g1, gp1 = _tc_prep(degp, m1)

    s1 = _sc_scatter_packed(gp1, src, dst, zeros_hbm)
    g2, gp2 = _tc_layer(s1, g1, dinv, b1r, W2)
    s2 = _sc_scatter_packed(gp2, src, dst, zeros_hbm)
    g3, gp3 = _tc_layer(s2, g2, dinv, b2r, W3)
    s3 = _sc_scatter_packed(gp3, src, dst, zeros_hbm)
    h_pad, out_pad = _tc_final(s3, g3, dinv, b3r, Wc_pad, bc_pad)

    return out_pad[:N, :C], h_pad[:N]
